# baseline (speedup 1.0000x reference)
import jax
import jax.numpy as jnp
from jax import lax
from jax.experimental import pallas as pl
from jax.experimental.pallas import tpu as pltpu
from jax.experimental.pallas import tpu_sc as plsc

N_FIELDS = 26
VOCAB = 100000
EMBED_DIM = 32
BATCH = 16384

NC, NS = 2, 16
NW = NC * NS
BPW = BATCH // NW
CHUNK = 64
NCHUNK = BPW // CHUNK
GROUP = 128
GPC = CHUNK * N_FIELDS // GROUP
HALF = EMBED_DIM // 2


def _emb_body(tables_hbm, idx_hbm, out_hbm, idx_v, rows_v, acc_v, sem0, sem1):
    wid = lax.axis_index("s") * NC + lax.axis_index("c")
    sems = (sem0, sem1)

    def issue(c):
        buf = c % 2
        pltpu.sync_copy(idx_hbm.at[wid, c], idx_v.at[buf])
        descs = []
        for j in range(GPC):
            descs.append(pltpu.async_copy(
                tables_hbm.at[idx_v.at[buf, j]],
                rows_v.at[buf, pl.ds(j * GROUP, GROUP)],
                sems[buf],
            ))
        return descs

    def compute(c):
        buf = c % 2

        def body(b, carry):
            r0 = b * N_FIELDS
            a0 = rows_v[buf, r0, pl.ds(0, HALF)]
            a1 = rows_v[buf, r0, pl.ds(HALF, HALF)]
            for f in range(1, N_FIELDS):
                a0 = a0 + rows_v[buf, r0 + f, pl.ds(0, HALF)]
                a1 = a1 + rows_v[buf, r0 + f, pl.ds(HALF, HALF)]
            acc_v[b, pl.ds(0, HALF)] = a0
            acc_v[b, pl.ds(HALF, HALF)] = a1
            return carry

        lax.fori_loop(0, CHUNK, body, 0)
        pltpu.sync_copy(acc_v, out_hbm.at[pl.ds(wid * BPW + c * CHUNK, CHUNK)])

    descs = issue(0)
    for c in range(NCHUNK):
        next_descs = issue(c + 1) if c + 1 < NCHUNK else None
        for d in descs:
            d.wait()
        compute(c)
        descs = next_descs


def kernel(x_categorical, tables):
    offs = jnp.arange(N_FIELDS, dtype=jnp.int32) * VOCAB
    idx = (x_categorical + offs[None, :]).reshape(NW, NCHUNK, GPC, GROUP)
    tab = tables.reshape(N_FIELDS * VOCAB, EMBED_DIM)

    run = pl.kernel(
        _emb_body,
        out_type=jax.ShapeDtypeStruct((BATCH, EMBED_DIM), jnp.float32),
        mesh=plsc.VectorSubcoreMesh(
            core_axis_name="c", subcore_axis_name="s",
            num_cores=NC, num_subcores=NS),
        scratch_types=[
            pltpu.VMEM((2, GPC, GROUP), jnp.int32),
            pltpu.VMEM((2, CHUNK * N_FIELDS, EMBED_DIM), jnp.float32),
            pltpu.VMEM((CHUNK, EMBED_DIM), jnp.float32),
            pltpu.SemaphoreType.DMA,
            pltpu.SemaphoreType.DMA,
        ],
        compiler_params=pltpu.CompilerParams(use_tc_tiling_on_sc=False),
    )
    return run(tab, idx)
